# baseline (speedup 1.0000x reference)
import functools

import jax
import jax.numpy as jnp
from jax import lax
from jax.experimental import pallas as pl
from jax.experimental.pallas import tpu as pltpu
from jax.experimental.pallas import tpu_sc as plsc

_NT = 26
_V = 100000
_D = 32
_B = 1024
_H = 20

_VT = 782
_TPC = 13
_TSTRIDE = 25024
_NLINES = _NT * _TSTRIDE
_UPT = 49
_NU = _TPC * _UPT
_CHUNK_BAGS = 8
_CHUNK_IDX = _CHUNK_BAGS * _H
_NCHUNK = _TPC * (64 // _CHUNK_BAGS)
_HALF = _B * _TPC * _D
_INV_H = 1.0 / _H


def _sc_body(tab, idx_hbm, relaid, out1d, src, trans, idx3, rlist0, rlist1, rov3, rows, outc,
             ssem, wsem, isem, gsem):
    core = lax.axis_index("c")
    s = lax.axis_index("s")
    t_base = core * _TPC
    it = lax.iota(jnp.int32, 16)

    def vtake(vec, lane):
        dn = lax.GatherDimensionNumbers(
            offset_dims=(), collapsed_slice_dims=(0,), start_index_map=(0,))
        return lax.gather(
            vec, lane.reshape(16, 1), dn, (1,),
            mode=lax.GatherScatterMode.PROMISE_IN_BOUNDS)

    rbase = [(vvb * 16 + it) >> 2 for vvb in range(8)]
    cbase = [((vvb * 16 + it) & 3) * 32 for vvb in range(8)]

    def unit_params(u):
        t_loc = u // _UPT
        c = s + 16 * (u % _UPT)
        return t_base + t_loc, c

    def fire_stage(u):
        t, c = unit_params(u)

        @pl.when(c < _VT)
        def _():
            buf = u % 2

            coff = pl.multiple_of(c * 128, 128)

            @pl.when(c < _VT - 1)
            def _():
                for j in range(4):
                    pltpu.async_copy(
                        tab.at[t, pl.ds(8 * j, 8), pl.ds(coff, 128)],
                        src.at[buf].at[pl.ds(8 * j, 8)], ssem)

            @pl.when(c == _VT - 1)
            def _():
                for j in range(4):
                    pltpu.async_copy(
                        tab.at[t, pl.ds(8 * j, 8), pl.ds(coff, 32)],
                        src.at[buf].at[pl.ds(8 * j, 8), pl.ds(0, 32)], ssem)

    def drain_stage(u):
        _, c = unit_params(u)

        @pl.when(c < _VT - 1)
        def _():
            for j in range(4):
                pltpu.make_async_copy(
                    tab.at[0, pl.ds(8 * j, 8), pl.ds(0, 128)],
                    src.at[u % 2].at[pl.ds(8 * j, 8)], ssem).wait()

        @pl.when(c == _VT - 1)
        def _():
            for j in range(4):
                pltpu.make_async_copy(
                    tab.at[0, pl.ds(8 * j, 8), pl.ds(0, 32)],
                    src.at[u % 2].at[pl.ds(8 * j, 8), pl.ds(0, 32)], ssem).wait()

    def drain_write(u):
        _, c = unit_params(u)

        @pl.when(c < _VT)
        def _():
            pltpu.make_async_copy(
                trans.at[u % 2], relaid.at[pl.ds(0, 32)], wsem).wait()

    def transpose_unit(u):
        t, c = unit_params(u)
        buf = u % 2

        @pl.when(c < _VT - 1)
        def _():
            for vvb in range(8):
                for d in range(32):
                    x = src[buf, d, pl.ds(vvb * 16, 16)]
                    plsc.store_scatter(trans.at[buf], [rbase[vvb], cbase[vvb] + d], x)

        @pl.when(c == _VT - 1)
        def _():
            for vvb in range(2):
                for d in range(32):
                    x = src[buf, d, pl.ds(vvb * 16, 16)]
                    plsc.store_scatter(trans.at[buf], [rbase[vvb], cbase[vvb] + d], x)

        @pl.when(c < _VT)
        def _():
            roff = pl.multiple_of(t * _TSTRIDE + c * 32, 8)
            pltpu.async_copy(trans.at[buf], relaid.at[pl.ds(roff, 32)], wsem)

    fire_stage(0)

    def p1_body(u, carry):
        drain_stage(u)
        fire_stage(u + 1)

        @pl.when(u >= 1)
        def _():
            drain_write(u - 1)

        transpose_unit(u)
        return carry

    lax.fori_loop(0, _NU - 1, p1_body, 0)
    drain_stage(_NU - 1)
    drain_write(_NU - 2)
    transpose_unit(_NU - 1)
    drain_write(_NU - 1)

    plsc.subcore_barrier()

    def fire_idx(n):
        @pl.when(n < _NCHUNK)
        def _():
            t_loc = n // 8
            grp = n % 8
            row = (t_base + t_loc) * 128 + s * 8 + grp
            pltpu.async_copy(idx_hbm.at[row], idx3.at[n % 3], isem)

    def prep_and_fire_gather(n):
        @pl.when(n < _NCHUNK)
        def _():
            pltpu.make_async_copy(idx_hbm.at[0], idx3.at[n % 3], isem).wait()
            t_loc = n // 8
            pad = (t_base + t_loc) * 24
            b2 = n % 2

            def emit(rl):
                for g in range(16):
                    v = idx3[n % 3, pl.ds(g * 16, 16)]
                    rl[pl.ds(g * 16, 16)] = (v >> 2) + pad
                    rov3[b2, g, :] = (v & 3) * 32

            @pl.when(b2 == 0)
            def _():
                emit(rlist0)
                pltpu.async_copy(relaid.at[rlist0], rows.at[0], gsem)

            @pl.when(b2 == 1)
            def _():
                emit(rlist1)
                pltpu.async_copy(relaid.at[rlist1], rows.at[1], gsem)

    def drain_gather():
        pltpu.make_async_copy(relaid.at[pl.ds(0, 256)], rows.at[0], gsem).wait()

    def reduce_chunk(n):
        t_loc = n // 8
        grp = n % 8
        b2 = n % 2
        nbuf = jnp.full((16,), 0, jnp.int32) + b2

        def bag_body(j, carry):
            acc_lo = jnp.zeros((16,), jnp.float32)
            acc_hi = jnp.zeros((16,), jnp.float32)
            for h in range(_H):
                k = j * _H + h
                rv = rov3[b2, k // 16]
                o = vtake(rv, jnp.full((16,), 0, jnp.int32) + (k % 16))
                rowv = jnp.full((16,), 0, jnp.int32) + k
                col = o + it
                acc_lo = acc_lo + plsc.load_gather(rows, [nbuf, rowv, col])
                acc_hi = acc_hi + plsc.load_gather(rows, [nbuf, rowv, col + 16])
            base = (grp * 8 + j) * (_TPC * _D) + t_loc * _D
            plsc.store_scatter(outc, [base + it], acc_lo * _INV_H)
            plsc.store_scatter(outc, [base + 16 + it], acc_hi * _INV_H)
            return carry

        lax.fori_loop(0, _CHUNK_BAGS, bag_body, 0)

    fire_idx(0)
    fire_idx(1)
    prep_and_fire_gather(0)

    def p2_body(n, carry):
        fire_idx(n + 2)
        prep_and_fire_gather(n + 1)
        drain_gather()
        reduce_chunk(n)
        return carry

    lax.fori_loop(0, _NCHUNK, p2_body, 0)
    ooff = pl.multiple_of(core * _HALF + s * (_HALF // 16), 8)
    pltpu.sync_copy(outc, out1d.at[pl.ds(ooff, _HALF // 16)])


_sc_call = functools.partial(
    pl.kernel,
    out_type=(
        jax.ShapeDtypeStruct((_NLINES, 128), jnp.float32),
        jax.ShapeDtypeStruct((2 * _HALF,), jnp.float32),
    ),
    mesh=plsc.VectorSubcoreMesh(core_axis_name="c", subcore_axis_name="s"),
    scratch_types=[
        pltpu.VMEM((2, 32, 128), jnp.float32),
        pltpu.VMEM((2, 32, 128), jnp.float32),
        pltpu.VMEM((3, 256), jnp.int32),
        pltpu.VMEM((256,), jnp.int32),
        pltpu.VMEM((256,), jnp.int32),
        pltpu.VMEM((2, 16, 16), jnp.int32),
        pltpu.VMEM((2, 256, 128), jnp.float32),
        pltpu.VMEM((_B // 16 * _TPC * _D,), jnp.float32),
        pltpu.SemaphoreType.DMA,
        pltpu.SemaphoreType.DMA,
        pltpu.SemaphoreType.DMA,
        pltpu.SemaphoreType.DMA,
    ],
    compiler_params=pltpu.CompilerParams(use_tc_tiling_on_sc=True, needs_layout_passes=False),
)(_sc_body)


@jax.jit
def kernel(indices, offsets, tables):
    del offsets
    tab_t = tables.transpose(0, 2, 1)
    idx2d = jnp.pad(indices.reshape(_NT * 128, _CHUNK_IDX), ((0, 0), (0, 256 - _CHUNK_IDX)))
    _, out1d = _sc_call(tab_t, idx2d)
    o = out1d.reshape(2, _B, _TPC * _D)
    return jnp.concatenate([o[0], o[1]], axis=1)


# speedup vs baseline: 2.8768x; 2.8768x over previous
import functools

import jax
import jax.numpy as jnp
from jax import lax
from jax.experimental import pallas as pl
from jax.experimental.pallas import tpu as pltpu
from jax.experimental.pallas import tpu_sc as plsc

_NT = 26
_V = 100000
_D = 32
_B = 1024
_H = 20

_VT = 782
_TPC = 13
_TSTRIDE = 25024
_NLINES = _NT * _TSTRIDE
_UPT = 49
_NU = _TPC * _UPT
_CHUNK_BAGS = 8
_CHUNK_IDX = _CHUNK_BAGS * _H
_NCHUNK = _TPC * (64 // _CHUNK_BAGS)
_HALF = _B * _TPC * _D
_INV_H = 1.0 / _H


def _sc_body(tab, idx_hbm, relaid, out1d, src, trans, idx3, rlist0, rlist1, rov3, rows, outc,
             ssem, wsem, isem, gsem):
    core = lax.axis_index("c")
    s = lax.axis_index("s")
    t_base = core * _TPC
    it = lax.iota(jnp.int32, 16)

    def vtake(vec, lane):
        dn = lax.GatherDimensionNumbers(
            offset_dims=(), collapsed_slice_dims=(0,), start_index_map=(0,))
        return lax.gather(
            vec, lane.reshape(16, 1), dn, (1,),
            mode=lax.GatherScatterMode.PROMISE_IN_BOUNDS)

    rbase = [(vvb * 16 + it) >> 2 for vvb in range(8)]
    cbase = [((vvb * 16 + it) & 3) * 32 for vvb in range(8)]

    def unit_params(u):
        t_loc = u // _UPT
        c = s + 16 * (u % _UPT)
        return t_base + t_loc, c

    def fire_stage(u):
        t, c = unit_params(u)

        @pl.when(c < _VT)
        def _():
            buf = u % 2

            coff = pl.multiple_of(c * 128, 128)

            @pl.when(c < _VT - 1)
            def _():
                pltpu.async_copy(
                    tab.at[t, pl.ds(0, 32), pl.ds(coff, 128)],
                    src.at[buf], ssem)

            @pl.when(c == _VT - 1)
            def _():
                for j in range(4):
                    pltpu.async_copy(
                        tab.at[t, pl.ds(8 * j, 8), pl.ds(coff, 32)],
                        src.at[buf].at[pl.ds(8 * j, 8), pl.ds(0, 32)], ssem)

    def drain_stage(u):
        _, c = unit_params(u)

        @pl.when(c < _VT - 1)
        def _():
            pltpu.make_async_copy(
                tab.at[0, pl.ds(0, 32), pl.ds(0, 128)],
                src.at[u % 2], ssem).wait()

        @pl.when(c == _VT - 1)
        def _():
            for j in range(4):
                pltpu.make_async_copy(
                    tab.at[0, pl.ds(8 * j, 8), pl.ds(0, 32)],
                    src.at[u % 2].at[pl.ds(8 * j, 8), pl.ds(0, 32)], ssem).wait()

    def drain_write(u):
        _, c = unit_params(u)

        @pl.when(c < _VT)
        def _():
            pltpu.make_async_copy(
                trans.at[u % 2], relaid.at[pl.ds(0, 32)], wsem).wait()

    def transpose_unit(u):
        t, c = unit_params(u)
        buf = u % 2

        @pl.when(c < _VT - 1)
        def _():
            for vvb in range(8):
                for d in range(32):
                    x = src[buf, d, pl.ds(vvb * 16, 16)]
                    plsc.store_scatter(trans.at[buf], [rbase[vvb], cbase[vvb] + d], x)

        @pl.when(c == _VT - 1)
        def _():
            for vvb in range(2):
                for d in range(32):
                    x = src[buf, d, pl.ds(vvb * 16, 16)]
                    plsc.store_scatter(trans.at[buf], [rbase[vvb], cbase[vvb] + d], x)

        @pl.when(c < _VT)
        def _():
            roff = pl.multiple_of(t * _TSTRIDE + c * 32, 8)
            pltpu.async_copy(trans.at[buf], relaid.at[pl.ds(roff, 32)], wsem)

    fire_stage(0)

    def p1_body(u, carry):
        drain_stage(u)
        fire_stage(u + 1)

        @pl.when(u >= 1)
        def _():
            drain_write(u - 1)

        transpose_unit(u)
        return carry

    lax.fori_loop(0, _NU - 1, p1_body, 0)
    drain_stage(_NU - 1)
    drain_write(_NU - 2)
    transpose_unit(_NU - 1)
    drain_write(_NU - 1)

    plsc.subcore_barrier()

    def fire_idx(n):
        @pl.when(n < _NCHUNK)
        def _():
            t_loc = n // 8
            grp = n % 8
            row = (t_base + t_loc) * 128 + s * 8 + grp
            pltpu.async_copy(idx_hbm.at[row], idx3.at[n % 3], isem)

    def prep_and_fire_gather(n):
        @pl.when(n < _NCHUNK)
        def _():
            pltpu.make_async_copy(idx_hbm.at[0], idx3.at[n % 3], isem).wait()
            t_loc = n // 8
            pad = (t_base + t_loc) * 24
            b2 = n % 2

            def emit(rl):
                for g in range(16):
                    v = idx3[n % 3, pl.ds(g * 16, 16)]
                    rl[pl.ds(g * 16, 16)] = (v >> 2) + pad
                    rov3[b2, g, :] = (v & 3) * 32

            @pl.when(b2 == 0)
            def _():
                emit(rlist0)
                pltpu.async_copy(relaid.at[rlist0], rows.at[0], gsem)

            @pl.when(b2 == 1)
            def _():
                emit(rlist1)
                pltpu.async_copy(relaid.at[rlist1], rows.at[1], gsem)

    def drain_gather():
        pltpu.make_async_copy(relaid.at[pl.ds(0, 256)], rows.at[0], gsem).wait()

    def reduce_chunk(n):
        t_loc = n // 8
        grp = n % 8
        b2 = n % 2
        nbuf = jnp.full((16,), 0, jnp.int32) + b2

        def bag_body(j, carry):
            acc_lo = jnp.zeros((16,), jnp.float32)
            acc_hi = jnp.zeros((16,), jnp.float32)
            for h in range(_H):
                k = j * _H + h
                rv = rov3[b2, k // 16]
                o = vtake(rv, jnp.full((16,), 0, jnp.int32) + (k % 16))
                rowv = jnp.full((16,), 0, jnp.int32) + k
                col = o + it
                acc_lo = acc_lo + plsc.load_gather(rows, [nbuf, rowv, col])
                acc_hi = acc_hi + plsc.load_gather(rows, [nbuf, rowv, col + 16])
            base = (grp * 8 + j) * (_TPC * _D) + t_loc * _D
            plsc.store_scatter(outc, [base + it], acc_lo * _INV_H)
            plsc.store_scatter(outc, [base + 16 + it], acc_hi * _INV_H)
            return carry

        lax.fori_loop(0, _CHUNK_BAGS, bag_body, 0)

    fire_idx(0)
    fire_idx(1)
    prep_and_fire_gather(0)

    def p2_body(n, carry):
        fire_idx(n + 2)
        prep_and_fire_gather(n + 1)
        drain_gather()
        reduce_chunk(n)
        return carry

    lax.fori_loop(0, _NCHUNK, p2_body, 0)
    ooff = pl.multiple_of(core * _HALF + s * (_HALF // 16), 8)
    pltpu.sync_copy(outc, out1d.at[pl.ds(ooff, _HALF // 16)])


_sc_call = functools.partial(
    pl.kernel,
    out_type=(
        jax.ShapeDtypeStruct((_NLINES, 128), jnp.float32),
        jax.ShapeDtypeStruct((2 * _HALF,), jnp.float32),
    ),
    mesh=plsc.VectorSubcoreMesh(core_axis_name="c", subcore_axis_name="s"),
    scratch_types=[
        pltpu.VMEM((2, 32, 128), jnp.float32),
        pltpu.VMEM((2, 32, 128), jnp.float32),
        pltpu.VMEM((3, 256), jnp.int32),
        pltpu.VMEM((256,), jnp.int32),
        pltpu.VMEM((256,), jnp.int32),
        pltpu.VMEM((2, 16, 16), jnp.int32),
        pltpu.VMEM((2, 256, 128), jnp.float32),
        pltpu.VMEM((_B // 16 * _TPC * _D,), jnp.float32),
        pltpu.SemaphoreType.DMA,
        pltpu.SemaphoreType.DMA,
        pltpu.SemaphoreType.DMA,
        pltpu.SemaphoreType.DMA,
    ],
    compiler_params=pltpu.CompilerParams(use_tc_tiling_on_sc=True, needs_layout_passes=False),
)(_sc_body)


@jax.jit
def kernel(indices, offsets, tables):
    del offsets
    tab_t = tables.transpose(0, 2, 1)
    base = (jnp.arange(_NT * 128, dtype=jnp.int32) // 128) * _V
    spread = jnp.arange(256 - _CHUNK_IDX, dtype=jnp.int32) * 977
    pad_blk = base[:, None] + spread[None, :]
    idx2d = jnp.concatenate(
        [indices.reshape(_NT * 128, _CHUNK_IDX), pad_blk], axis=1)
    _, out1d = _sc_call(tab_t, idx2d)
    o = out1d.reshape(2, _B, _TPC * _D)
    return jnp.concatenate([o[0], o[1]], axis=1)


# speedup vs baseline: 2.8769x; 1.0000x over previous
import functools

import jax
import jax.numpy as jnp
from jax import lax
from jax.experimental import pallas as pl
from jax.experimental.pallas import tpu as pltpu
from jax.experimental.pallas import tpu_sc as plsc

_NT = 26
_V = 100000
_D = 32
_B = 1024
_H = 20

_VT = 782
_TPC = 13
_TSTRIDE = 25024
_NLINES = _NT * _TSTRIDE
_UPT = 49
_NU = _TPC * _UPT
_CHUNK_BAGS = 8
_CHUNK_IDX = _CHUNK_BAGS * _H
_NCHUNK = _TPC * (64 // _CHUNK_BAGS)
_HALF = _B * _TPC * _D
_INV_H = 1.0 / _H


def _sc_body(tab, idx_hbm, relaid, out1d, src, trans, idx3, rlist0, rlist1, rov3, rows, outc,
             ssem, wsem, isem, gsem):
    core = lax.axis_index("c")
    s = lax.axis_index("s")
    t_base = core * _TPC
    it = lax.iota(jnp.int32, 16)

    def vtake(vec, lane):
        dn = lax.GatherDimensionNumbers(
            offset_dims=(), collapsed_slice_dims=(0,), start_index_map=(0,))
        return lax.gather(
            vec, lane.reshape(16, 1), dn, (1,),
            mode=lax.GatherScatterMode.PROMISE_IN_BOUNDS)

    rbase = [(vvb * 16 + it) >> 2 for vvb in range(8)]
    cbase = [((vvb * 16 + it) & 3) * 32 for vvb in range(8)]

    def unit_params(u):
        t_loc = u // _UPT
        c = s + 16 * (u % _UPT)
        return t_base + t_loc, c

    def fire_stage(u):
        t, c = unit_params(u)

        @pl.when((c < _VT) & (u < _NU))
        def _():
            buf = u % 3

            coff = pl.multiple_of(c * 128, 128)

            @pl.when(c < _VT - 1)
            def _():
                pltpu.async_copy(
                    tab.at[t, pl.ds(0, 32), pl.ds(coff, 128)],
                    src.at[buf], ssem)

            @pl.when(c == _VT - 1)
            def _():
                for j in range(4):
                    pltpu.async_copy(
                        tab.at[t, pl.ds(8 * j, 8), pl.ds(coff, 32)],
                        src.at[buf].at[pl.ds(8 * j, 8), pl.ds(0, 32)], ssem)

    def drain_stage(u):
        _, c = unit_params(u)

        @pl.when(c < _VT - 1)
        def _():
            pltpu.make_async_copy(
                tab.at[0, pl.ds(0, 32), pl.ds(0, 128)],
                src.at[u % 3], ssem).wait()

        @pl.when(c == _VT - 1)
        def _():
            for j in range(4):
                pltpu.make_async_copy(
                    tab.at[0, pl.ds(8 * j, 8), pl.ds(0, 32)],
                    src.at[u % 3].at[pl.ds(8 * j, 8), pl.ds(0, 32)], ssem).wait()

    def drain_write(u):
        _, c = unit_params(u)

        @pl.when(c < _VT)
        def _():
            pltpu.make_async_copy(
                trans.at[u % 2], relaid.at[pl.ds(0, 32)], wsem).wait()

    def transpose_unit(u):
        t, c = unit_params(u)
        buf = u % 3
        wbuf = u % 2

        @pl.when(c < _VT - 1)
        def _():
            for vvb in range(8):
                for d in range(32):
                    x = src[buf, d, pl.ds(vvb * 16, 16)]
                    plsc.store_scatter(trans.at[wbuf], [rbase[vvb], cbase[vvb] + d], x)

        @pl.when(c == _VT - 1)
        def _():
            for vvb in range(2):
                for d in range(32):
                    x = src[buf, d, pl.ds(vvb * 16, 16)]
                    plsc.store_scatter(trans.at[wbuf], [rbase[vvb], cbase[vvb] + d], x)

        @pl.when(c < _VT)
        def _():
            roff = pl.multiple_of(t * _TSTRIDE + c * 32, 8)
            pltpu.async_copy(trans.at[wbuf], relaid.at[pl.ds(roff, 32)], wsem)

    fire_stage(0)
    fire_stage(1)

    def p1_body(u, carry):
        drain_stage(u)
        fire_stage(u + 2)

        @pl.when(u >= 1)
        def _():
            drain_write(u - 1)

        transpose_unit(u)
        return carry

    lax.fori_loop(0, _NU - 1, p1_body, 0)
    drain_stage(_NU - 1)
    drain_write(_NU - 2)
    transpose_unit(_NU - 1)
    drain_write(_NU - 1)

    plsc.subcore_barrier()

    def fire_idx(n):
        @pl.when(n < _NCHUNK)
        def _():
            t_loc = n // 8
            grp = n % 8
            row = (t_base + t_loc) * 128 + s * 8 + grp
            pltpu.async_copy(idx_hbm.at[row], idx3.at[n % 3], isem)

    def prep_and_fire_gather(n):
        @pl.when(n < _NCHUNK)
        def _():
            pltpu.make_async_copy(idx_hbm.at[0], idx3.at[n % 3], isem).wait()
            t_loc = n // 8
            pad = (t_base + t_loc) * 24
            b2 = n % 2

            def emit(rl):
                for g in range(16):
                    v = idx3[n % 3, pl.ds(g * 16, 16)]
                    rl[pl.ds(g * 16, 16)] = (v >> 2) + pad
                    rov3[b2, g, :] = (v & 3) * 32

            @pl.when(b2 == 0)
            def _():
                emit(rlist0)
                pltpu.async_copy(relaid.at[rlist0], rows.at[0], gsem)

            @pl.when(b2 == 1)
            def _():
                emit(rlist1)
                pltpu.async_copy(relaid.at[rlist1], rows.at[1], gsem)

    def drain_gather():
        pltpu.make_async_copy(relaid.at[pl.ds(0, 256)], rows.at[0], gsem).wait()

    def reduce_chunk(n):
        t_loc = n // 8
        grp = n % 8
        b2 = n % 2
        nbuf = jnp.full((16,), 0, jnp.int32) + b2

        def bag_body(j, carry):
            acc_lo = jnp.zeros((16,), jnp.float32)
            acc_hi = jnp.zeros((16,), jnp.float32)
            for h in range(_H):
                k = j * _H + h
                rv = rov3[b2, k // 16]
                o = vtake(rv, jnp.full((16,), 0, jnp.int32) + (k % 16))
                rowv = jnp.full((16,), 0, jnp.int32) + k
                col = o + it
                acc_lo = acc_lo + plsc.load_gather(rows, [nbuf, rowv, col])
                acc_hi = acc_hi + plsc.load_gather(rows, [nbuf, rowv, col + 16])
            base = (grp * 8 + j) * (_TPC * _D) + t_loc * _D
            plsc.store_scatter(outc, [base + it], acc_lo * _INV_H)
            plsc.store_scatter(outc, [base + 16 + it], acc_hi * _INV_H)
            return carry

        lax.fori_loop(0, _CHUNK_BAGS, bag_body, 0)

    fire_idx(0)
    fire_idx(1)
    prep_and_fire_gather(0)

    def p2_body(n, carry):
        fire_idx(n + 2)
        prep_and_fire_gather(n + 1)
        drain_gather()
        reduce_chunk(n)
        return carry

    lax.fori_loop(0, _NCHUNK, p2_body, 0)
    ooff = pl.multiple_of(core * _HALF + s * (_HALF // 16), 8)
    pltpu.sync_copy(outc, out1d.at[pl.ds(ooff, _HALF // 16)])


_sc_call = functools.partial(
    pl.kernel,
    out_type=(
        jax.ShapeDtypeStruct((_NLINES, 128), jnp.float32),
        jax.ShapeDtypeStruct((2 * _HALF,), jnp.float32),
    ),
    mesh=plsc.VectorSubcoreMesh(core_axis_name="c", subcore_axis_name="s"),
    scratch_types=[
        pltpu.VMEM((3, 32, 128), jnp.float32),
        pltpu.VMEM((2, 32, 128), jnp.float32),
        pltpu.VMEM((3, 256), jnp.int32),
        pltpu.VMEM((256,), jnp.int32),
        pltpu.VMEM((256,), jnp.int32),
        pltpu.VMEM((2, 16, 16), jnp.int32),
        pltpu.VMEM((2, 256, 128), jnp.float32),
        pltpu.VMEM((_B // 16 * _TPC * _D,), jnp.float32),
        pltpu.SemaphoreType.DMA,
        pltpu.SemaphoreType.DMA,
        pltpu.SemaphoreType.DMA,
        pltpu.SemaphoreType.DMA,
    ],
    compiler_params=pltpu.CompilerParams(use_tc_tiling_on_sc=True, needs_layout_passes=False),
)(_sc_body)


@jax.jit
def kernel(indices, offsets, tables):
    del offsets
    tab_t = tables.transpose(0, 2, 1)
    base = (jnp.arange(_NT * 128, dtype=jnp.int32) // 128) * _V
    spread = jnp.arange(256 - _CHUNK_IDX, dtype=jnp.int32) * 977
    pad_blk = base[:, None] + spread[None, :]
    idx2d = jnp.concatenate(
        [indices.reshape(_NT * 128, _CHUNK_IDX), pad_blk], axis=1)
    _, out1d = _sc_call(tab_t, idx2d)
    o = out1d.reshape(2, _B, _TPC * _D)
    return jnp.concatenate([o[0], o[1]], axis=1)


# speedup vs baseline: 4.1541x; 1.4440x over previous
import functools

import jax
import jax.numpy as jnp
from jax import lax
from jax.experimental import pallas as pl
from jax.experimental.pallas import tpu as pltpu
from jax.experimental.pallas import tpu_sc as plsc

_NT = 26
_V = 100000
_D = 32
_B = 1024
_H = 20

_VT = 782
_TPC = 13
_TSTRIDE = 25024
_NLINES = _NT * _TSTRIDE
_UPT = 49
_NU = _TPC * _UPT
_CHUNK_BAGS = 8
_CHUNK_IDX = _CHUNK_BAGS * _H
_NCHUNK = _TPC * (64 // _CHUNK_BAGS)
_HALF = _B * _TPC * _D
_INV_H = 1.0 / _H


def _sc_body(tab, idx_hbm, relaid, out1d, src, trans, idx3, rlist0, rlist1, rov3, rows, outc,
             ssem, wsem, isem, gsem):
    core = lax.axis_index("c")
    s = lax.axis_index("s")
    t_base = core * _TPC
    it = lax.iota(jnp.int32, 16)

    def vtake(vec, lane):
        dn = lax.GatherDimensionNumbers(
            offset_dims=(), collapsed_slice_dims=(0,), start_index_map=(0,))
        return lax.gather(
            vec, lane.reshape(16, 1), dn, (1,),
            mode=lax.GatherScatterMode.PROMISE_IN_BOUNDS)

    rbase = [(vvb * 16 + it) >> 2 for vvb in range(8)]
    cbase = [((vvb * 16 + it) & 3) * 32 for vvb in range(8)]

    def unit_params(u):
        t_loc = u // _UPT
        c = s + 16 * (u % _UPT)
        return t_base + t_loc, c

    def fire_stage(u):
        t, c = unit_params(u)

        @pl.when((c < _VT) & (u < _NU))
        def _():
            buf = u % 3

            coff = pl.multiple_of(c * 128, 128)

            @pl.when(c < _VT - 1)
            def _():
                pltpu.async_copy(
                    tab.at[t, pl.ds(0, 32), pl.ds(coff, 128)],
                    src.at[buf], ssem)

            @pl.when(c == _VT - 1)
            def _():
                for j in range(4):
                    pltpu.async_copy(
                        tab.at[t, pl.ds(8 * j, 8), pl.ds(coff, 32)],
                        src.at[buf].at[pl.ds(8 * j, 8), pl.ds(0, 32)], ssem)

    def drain_stage(u):
        _, c = unit_params(u)

        @pl.when(c < _VT - 1)
        def _():
            pltpu.make_async_copy(
                tab.at[0, pl.ds(0, 32), pl.ds(0, 128)],
                src.at[u % 3], ssem).wait()

        @pl.when(c == _VT - 1)
        def _():
            for j in range(4):
                pltpu.make_async_copy(
                    tab.at[0, pl.ds(8 * j, 8), pl.ds(0, 32)],
                    src.at[u % 3].at[pl.ds(8 * j, 8), pl.ds(0, 32)], ssem).wait()

    def drain_write(u):
        _, c = unit_params(u)

        @pl.when(c < _VT)
        def _():
            pltpu.make_async_copy(
                trans.at[u % 2], relaid.at[pl.ds(0, 32)], wsem).wait()

    def transpose_unit(u):
        t, c = unit_params(u)
        buf = u % 3
        wbuf = u % 2

        @pl.when(c < _VT - 1)
        def _():
            @plsc.parallel_loop(0, 32, unroll=4)
            def _(d):
                for vvb in range(8):
                    x = src[buf, d, pl.ds(vvb * 16, 16)]
                    plsc.store_scatter(trans.at[wbuf], [rbase[vvb], cbase[vvb] + d], x)

        @pl.when(c == _VT - 1)
        def _():
            @plsc.parallel_loop(0, 32, unroll=4)
            def _(d):
                for vvb in range(2):
                    x = src[buf, d, pl.ds(vvb * 16, 16)]
                    plsc.store_scatter(trans.at[wbuf], [rbase[vvb], cbase[vvb] + d], x)

        @pl.when(c < _VT)
        def _():
            roff = pl.multiple_of(t * _TSTRIDE + c * 32, 8)
            pltpu.async_copy(trans.at[wbuf], relaid.at[pl.ds(roff, 32)], wsem)

    fire_stage(0)
    fire_stage(1)

    def p1_body(u, carry):
        drain_stage(u)
        fire_stage(u + 2)

        @pl.when(u >= 1)
        def _():
            drain_write(u - 1)

        transpose_unit(u)
        return carry

    lax.fori_loop(0, _NU - 1, p1_body, 0)
    drain_stage(_NU - 1)
    drain_write(_NU - 2)
    transpose_unit(_NU - 1)
    drain_write(_NU - 1)

    plsc.subcore_barrier()

    def fire_idx(n):
        @pl.when(n < _NCHUNK)
        def _():
            t_loc = n // 8
            grp = n % 8
            row = (t_base + t_loc) * 128 + s * 8 + grp
            pltpu.async_copy(idx_hbm.at[row], idx3.at[n % 3], isem)

    def prep_and_fire_gather(n):
        @pl.when(n < _NCHUNK)
        def _():
            pltpu.make_async_copy(idx_hbm.at[0], idx3.at[n % 3], isem).wait()
            t_loc = n // 8
            pad = (t_base + t_loc) * 24
            b2 = n % 2

            def emit(rl):
                for g in range(10):
                    v = idx3[n % 3, pl.ds(g * 16, 16)]
                    rl[pl.ds(g * 16, 16)] = (v >> 2) + pad
                    rov3[b2, g, :] = (v & 3) * 32

            @pl.when(b2 == 0)
            def _():
                emit(rlist0)
                pltpu.async_copy(relaid.at[rlist0], rows.at[0], gsem)

            @pl.when(b2 == 1)
            def _():
                emit(rlist1)
                pltpu.async_copy(relaid.at[rlist1], rows.at[1], gsem)

    def drain_gather():
        pltpu.make_async_copy(relaid.at[pl.ds(0, 160)], rows.at[0], gsem).wait()

    def reduce_chunk(n):
        t_loc = n // 8
        grp = n % 8
        b2 = n % 2
        nbuf = jnp.full((16,), 0, jnp.int32) + b2

        def bag_body(j, carry):
            acc_lo = jnp.zeros((16,), jnp.float32)
            acc_hi = jnp.zeros((16,), jnp.float32)
            for h in range(_H):
                k = j * _H + h
                rv = rov3[b2, k // 16]
                o = vtake(rv, jnp.full((16,), 0, jnp.int32) + (k % 16))
                rowv = jnp.full((16,), 0, jnp.int32) + k
                col = o + it
                acc_lo = acc_lo + plsc.load_gather(rows, [nbuf, rowv, col])
                acc_hi = acc_hi + plsc.load_gather(rows, [nbuf, rowv, col + 16])
            base = (grp * 8 + j) * (_TPC * _D) + t_loc * _D
            plsc.store_scatter(outc, [base + it], acc_lo * _INV_H)
            plsc.store_scatter(outc, [base + 16 + it], acc_hi * _INV_H)
            return carry

        lax.fori_loop(0, _CHUNK_BAGS, bag_body, 0)

    fire_idx(0)
    fire_idx(1)
    prep_and_fire_gather(0)

    def p2_body(n, carry):
        fire_idx(n + 2)
        prep_and_fire_gather(n + 1)
        drain_gather()
        reduce_chunk(n)
        return carry

    lax.fori_loop(0, _NCHUNK, p2_body, 0)
    ooff = pl.multiple_of(core * _HALF + s * (_HALF // 16), 8)
    pltpu.sync_copy(outc, out1d.at[pl.ds(ooff, _HALF // 16)])


_sc_call = functools.partial(
    pl.kernel,
    out_type=(
        jax.ShapeDtypeStruct((_NLINES, 128), jnp.float32),
        jax.ShapeDtypeStruct((2 * _HALF,), jnp.float32),
    ),
    mesh=plsc.VectorSubcoreMesh(core_axis_name="c", subcore_axis_name="s"),
    scratch_types=[
        pltpu.VMEM((3, 32, 128), jnp.float32),
        pltpu.VMEM((2, 32, 128), jnp.float32),
        pltpu.VMEM((3, 256), jnp.int32),
        pltpu.VMEM((160,), jnp.int32),
        pltpu.VMEM((160,), jnp.int32),
        pltpu.VMEM((2, 16, 16), jnp.int32),
        pltpu.VMEM((2, 160, 128), jnp.float32),
        pltpu.VMEM((_B // 16 * _TPC * _D,), jnp.float32),
        pltpu.SemaphoreType.DMA,
        pltpu.SemaphoreType.DMA,
        pltpu.SemaphoreType.DMA,
        pltpu.SemaphoreType.DMA,
    ],
    compiler_params=pltpu.CompilerParams(use_tc_tiling_on_sc=True, needs_layout_passes=False),
)(_sc_body)


@jax.jit
def kernel(indices, offsets, tables):
    del offsets
    tab_t = tables.transpose(0, 2, 1)
    base = (jnp.arange(_NT * 128, dtype=jnp.int32) // 128) * _V
    spread = jnp.arange(256 - _CHUNK_IDX, dtype=jnp.int32) * 977
    pad_blk = base[:, None] + spread[None, :]
    idx2d = jnp.concatenate(
        [indices.reshape(_NT * 128, _CHUNK_IDX), pad_blk], axis=1)
    _, out1d = _sc_call(tab_t, idx2d)
    o = out1d.reshape(2, _B, _TPC * _D)
    return jnp.concatenate([o[0], o[1]], axis=1)


# speedup vs baseline: 4.1593x; 1.0013x over previous
import functools

import jax
import jax.numpy as jnp
from jax import lax
from jax.experimental import pallas as pl
from jax.experimental.pallas import tpu as pltpu
from jax.experimental.pallas import tpu_sc as plsc

_NT = 26
_V = 100000
_D = 32
_B = 1024
_H = 20

_VT = 782
_TPC = 13
_TSTRIDE = 25024
_NLINES = _NT * _TSTRIDE
_UPT = 49
_NU = _TPC * _UPT
_CHUNK_BAGS = 8
_CHUNK_IDX = _CHUNK_BAGS * _H
_NCHUNK = _TPC * (64 // _CHUNK_BAGS)
_HALF = _B * _TPC * _D
_INV_H = 1.0 / _H


def _sc_body(tab, idx_hbm, relaid, out1d, src, trans, idx3, rlist0, rlist1, rov3, rows, outc,
             ssem, wsem, isem, gsem):
    core = lax.axis_index("c")
    s = lax.axis_index("s")
    t_base = core * _TPC
    it = lax.iota(jnp.int32, 16)

    def vtake(vec, lane):
        dn = lax.GatherDimensionNumbers(
            offset_dims=(), collapsed_slice_dims=(0,), start_index_map=(0,))
        return lax.gather(
            vec, lane.reshape(16, 1), dn, (1,),
            mode=lax.GatherScatterMode.PROMISE_IN_BOUNDS)

    rbase = [(vvb * 16 + it) >> 2 for vvb in range(8)]
    cbase = [((vvb * 16 + it) & 3) * 32 for vvb in range(8)]

    def unit_params(u):
        t_loc = u // _UPT
        c = s + 16 * (u % _UPT)
        return t_base + t_loc, c

    def fire_stage(u):
        t, c = unit_params(u)

        @pl.when((c < _VT) & (u < _NU))
        def _():
            buf = u % 3

            coff = pl.multiple_of(c * 128, 128)

            @pl.when(c < _VT - 1)
            def _():
                pltpu.async_copy(
                    tab.at[t, pl.ds(0, 32), pl.ds(coff, 128)],
                    src.at[buf], ssem)

            @pl.when(c == _VT - 1)
            def _():
                for j in range(4):
                    pltpu.async_copy(
                        tab.at[t, pl.ds(8 * j, 8), pl.ds(coff, 32)],
                        src.at[buf].at[pl.ds(8 * j, 8), pl.ds(0, 32)], ssem)

    def drain_stage(u):
        _, c = unit_params(u)

        @pl.when(c < _VT - 1)
        def _():
            pltpu.make_async_copy(
                tab.at[0, pl.ds(0, 32), pl.ds(0, 128)],
                src.at[u % 3], ssem).wait()

        @pl.when(c == _VT - 1)
        def _():
            for j in range(4):
                pltpu.make_async_copy(
                    tab.at[0, pl.ds(8 * j, 8), pl.ds(0, 32)],
                    src.at[u % 3].at[pl.ds(8 * j, 8), pl.ds(0, 32)], ssem).wait()

    def drain_write(u):
        _, c = unit_params(u)

        @pl.when(c < _VT)
        def _():
            pltpu.make_async_copy(
                trans.at[u % 2], relaid.at[pl.ds(0, 32)], wsem).wait()

    def transpose_unit(u):
        t, c = unit_params(u)
        buf = u % 3
        wbuf = u % 2

        @pl.when(c < _VT - 1)
        def _():
            @plsc.parallel_loop(0, 32, unroll=8)
            def _(d):
                for vvb in range(8):
                    x = src[buf, d, pl.ds(vvb * 16, 16)]
                    plsc.store_scatter(trans.at[wbuf], [rbase[vvb], cbase[vvb] + d], x)

        @pl.when(c == _VT - 1)
        def _():
            @plsc.parallel_loop(0, 32, unroll=8)
            def _(d):
                for vvb in range(2):
                    x = src[buf, d, pl.ds(vvb * 16, 16)]
                    plsc.store_scatter(trans.at[wbuf], [rbase[vvb], cbase[vvb] + d], x)

        @pl.when(c < _VT)
        def _():
            roff = pl.multiple_of(t * _TSTRIDE + c * 32, 8)
            pltpu.async_copy(trans.at[wbuf], relaid.at[pl.ds(roff, 32)], wsem)

    fire_stage(0)
    fire_stage(1)

    def p1_body(u, carry):
        drain_stage(u)
        fire_stage(u + 2)

        @pl.when(u >= 1)
        def _():
            drain_write(u - 1)

        transpose_unit(u)
        return carry

    lax.fori_loop(0, _NU - 1, p1_body, 0)
    drain_stage(_NU - 1)
    drain_write(_NU - 2)
    transpose_unit(_NU - 1)
    drain_write(_NU - 1)

    plsc.subcore_barrier()

    def fire_idx(n):
        @pl.when(n < _NCHUNK)
        def _():
            t_loc = n // 8
            grp = n % 8
            row = (t_base + t_loc) * 128 + s * 8 + grp
            pltpu.async_copy(idx_hbm.at[row], idx3.at[n % 3], isem)

    def prep_and_fire_gather(n):
        @pl.when(n < _NCHUNK)
        def _():
            pltpu.make_async_copy(idx_hbm.at[0], idx3.at[n % 3], isem).wait()
            t_loc = n // 8
            pad = (t_base + t_loc) * 24
            b2 = n % 2

            def emit(rl):
                for g in range(10):
                    v = idx3[n % 3, pl.ds(g * 16, 16)]
                    rl[pl.ds(g * 16, 16)] = (v >> 2) + pad
                    rov3[b2, g, :] = (v & 3) * 32

            @pl.when(b2 == 0)
            def _():
                emit(rlist0)
                pltpu.async_copy(relaid.at[rlist0], rows.at[0], gsem)

            @pl.when(b2 == 1)
            def _():
                emit(rlist1)
                pltpu.async_copy(relaid.at[rlist1], rows.at[1], gsem)

    def drain_gather():
        pltpu.make_async_copy(relaid.at[pl.ds(0, 160)], rows.at[0], gsem).wait()

    def reduce_chunk(n):
        t_loc = n // 8
        grp = n % 8
        b2 = n % 2
        nbuf = jnp.full((16,), 0, jnp.int32) + b2

        @plsc.parallel_loop(0, _CHUNK_BAGS, unroll=2)
        def bag_body(j):
            acc_lo = jnp.zeros((16,), jnp.float32)
            acc_hi = jnp.zeros((16,), jnp.float32)
            for h in range(_H):
                k = j * _H + h
                rv = rov3[b2, k // 16]
                o = vtake(rv, jnp.full((16,), 0, jnp.int32) + (k % 16))
                rowv = jnp.full((16,), 0, jnp.int32) + k
                col = o + it
                acc_lo = acc_lo + plsc.load_gather(rows, [nbuf, rowv, col])
                acc_hi = acc_hi + plsc.load_gather(rows, [nbuf, rowv, col + 16])
            base = (grp * 8 + j) * (_TPC * _D) + t_loc * _D
            plsc.store_scatter(outc, [base + it], acc_lo * _INV_H)
            plsc.store_scatter(outc, [base + 16 + it], acc_hi * _INV_H)

    fire_idx(0)
    fire_idx(1)
    prep_and_fire_gather(0)

    def p2_body(n, carry):
        fire_idx(n + 2)
        prep_and_fire_gather(n + 1)
        drain_gather()
        reduce_chunk(n)
        return carry

    lax.fori_loop(0, _NCHUNK, p2_body, 0)
    ooff = pl.multiple_of(core * _HALF + s * (_HALF // 16), 8)
    pltpu.sync_copy(outc, out1d.at[pl.ds(ooff, _HALF // 16)])


_sc_call = functools.partial(
    pl.kernel,
    out_type=(
        jax.ShapeDtypeStruct((_NLINES, 128), jnp.float32),
        jax.ShapeDtypeStruct((2 * _HALF,), jnp.float32),
    ),
    mesh=plsc.VectorSubcoreMesh(core_axis_name="c", subcore_axis_name="s"),
    scratch_types=[
        pltpu.VMEM((3, 32, 128), jnp.float32),
        pltpu.VMEM((2, 32, 128), jnp.float32),
        pltpu.VMEM((3, 256), jnp.int32),
        pltpu.VMEM((160,), jnp.int32),
        pltpu.VMEM((160,), jnp.int32),
        pltpu.VMEM((2, 16, 16), jnp.int32),
        pltpu.VMEM((2, 160, 128), jnp.float32),
        pltpu.VMEM((_B // 16 * _TPC * _D,), jnp.float32),
        pltpu.SemaphoreType.DMA,
        pltpu.SemaphoreType.DMA,
        pltpu.SemaphoreType.DMA,
        pltpu.SemaphoreType.DMA,
    ],
    compiler_params=pltpu.CompilerParams(use_tc_tiling_on_sc=True, needs_layout_passes=False),
)(_sc_body)


@jax.jit
def kernel(indices, offsets, tables):
    del offsets
    tab_t = tables.transpose(0, 2, 1)
    base = (jnp.arange(_NT * 128, dtype=jnp.int32) // 128) * _V
    spread = jnp.arange(256 - _CHUNK_IDX, dtype=jnp.int32) * 977
    pad_blk = base[:, None] + spread[None, :]
    idx2d = jnp.concatenate(
        [indices.reshape(_NT * 128, _CHUNK_IDX), pad_blk], axis=1)
    _, out1d = _sc_call(tab_t, idx2d)
    o = out1d.reshape(2, _B, _TPC * _D)
    return jnp.concatenate([o[0], o[1]], axis=1)


# speedup vs baseline: 4.4685x; 1.0743x over previous
import functools

import jax
import jax.numpy as jnp
from jax import lax
from jax.experimental import pallas as pl
from jax.experimental.pallas import tpu as pltpu
from jax.experimental.pallas import tpu_sc as plsc

_NT = 26
_V = 100000
_D = 32
_B = 1024
_H = 20

_VT = 782
_TPC = 13
_TSTRIDE = 25024
_NLINES = _NT * _TSTRIDE
_UPT = 49
_NU = _TPC * _UPT
_CHUNK_BAGS = 8
_CHUNK_IDX = _CHUNK_BAGS * _H
_NCHUNK = _TPC * (64 // _CHUNK_BAGS)
_HALF = _B * _TPC * _D
_INV_H = 1.0 / _H


def _sc_body(relaid, idx_hbm, out1d, idx3, rlist0, rlist1, rov3, rows, outc,
             isem, gsem):
    core = lax.axis_index("c")
    s = lax.axis_index("s")
    t_base = core * _TPC
    it = lax.iota(jnp.int32, 16)

    def vtake(vec, lane):
        dn = lax.GatherDimensionNumbers(
            offset_dims=(), collapsed_slice_dims=(0,), start_index_map=(0,))
        return lax.gather(
            vec, lane.reshape(16, 1), dn, (1,),
            mode=lax.GatherScatterMode.PROMISE_IN_BOUNDS)


    def fire_idx(n):
        @pl.when(n < _NCHUNK)
        def _():
            t_loc = n // 8
            grp = n % 8
            row = (t_base + t_loc) * 128 + s * 8 + grp
            pltpu.async_copy(idx_hbm.at[row], idx3.at[n % 3], isem)

    def prep_and_fire_gather(n):
        @pl.when(n < _NCHUNK)
        def _():
            pltpu.make_async_copy(idx_hbm.at[0], idx3.at[n % 3], isem).wait()
            t_loc = n // 8
            b2 = n % 2

            def emit(rl):
                for g in range(10):
                    v = idx3[n % 3, pl.ds(g * 16, 16)]
                    rl[pl.ds(g * 16, 16)] = v >> 2
                    rov3[b2, g, :] = (v & 3) * 32

            @pl.when(b2 == 0)
            def _():
                emit(rlist0)
                pltpu.async_copy(relaid.at[rlist0], rows.at[0], gsem)

            @pl.when(b2 == 1)
            def _():
                emit(rlist1)
                pltpu.async_copy(relaid.at[rlist1], rows.at[1], gsem)

    def drain_gather():
        pltpu.make_async_copy(relaid.at[pl.ds(0, 160)], rows.at[0], gsem).wait()

    def reduce_chunk(n):
        t_loc = n // 8
        grp = n % 8
        b2 = n % 2
        nbuf = jnp.full((16,), 0, jnp.int32) + b2

        def bag_body(j, carry):
            acc_lo = jnp.zeros((16,), jnp.float32)
            acc_hi = jnp.zeros((16,), jnp.float32)
            for h in range(_H):
                k = j * _H + h
                rv = rov3[b2, k // 16]
                o = vtake(rv, jnp.full((16,), 0, jnp.int32) + (k % 16))
                rowv = jnp.full((16,), 0, jnp.int32) + k
                col = o + it
                acc_lo = acc_lo + plsc.load_gather(rows, [nbuf, rowv, col])
                acc_hi = acc_hi + plsc.load_gather(rows, [nbuf, rowv, col + 16])
            base = (grp * 8 + j) * (_TPC * _D) + t_loc * _D
            plsc.store_scatter(outc, [base + it], acc_lo * _INV_H)
            plsc.store_scatter(outc, [base + 16 + it], acc_hi * _INV_H)
            return carry

        lax.fori_loop(0, _CHUNK_BAGS, bag_body, 0)

    fire_idx(0)
    fire_idx(1)
    prep_and_fire_gather(0)

    def p2_body(n, carry):
        fire_idx(n + 2)
        prep_and_fire_gather(n + 1)
        drain_gather()
        reduce_chunk(n)
        return carry

    lax.fori_loop(0, _NCHUNK, p2_body, 0)
    ooff = pl.multiple_of(core * _HALF + s * (_HALF // 16), 8)
    pltpu.sync_copy(outc, out1d.at[pl.ds(ooff, _HALF // 16)])


_sc_call = functools.partial(
    pl.kernel,
    out_type=jax.ShapeDtypeStruct((2 * _HALF,), jnp.float32),
    mesh=plsc.VectorSubcoreMesh(core_axis_name="c", subcore_axis_name="s"),
    scratch_types=[
        pltpu.VMEM((3, 256), jnp.int32),
        pltpu.VMEM((160,), jnp.int32),
        pltpu.VMEM((160,), jnp.int32),
        pltpu.VMEM((2, 16, 16), jnp.int32),
        pltpu.VMEM((2, 160, 128), jnp.float32),
        pltpu.VMEM((_B // 16 * _TPC * _D,), jnp.float32),
        pltpu.SemaphoreType.DMA,
        pltpu.SemaphoreType.DMA,
    ],
    compiler_params=pltpu.CompilerParams(use_tc_tiling_on_sc=False, needs_layout_passes=False),
)(_sc_body)


@jax.jit
def kernel(indices, offsets, tables):
    del offsets
    lines = tables.reshape(_NT * _V * _D // 128, 128)
    base = (jnp.arange(_NT * 128, dtype=jnp.int32) // 128) * _V
    spread = jnp.arange(256 - _CHUNK_IDX, dtype=jnp.int32) * 977
    pad_blk = base[:, None] + spread[None, :]
    idx2d = jnp.concatenate(
        [indices.reshape(_NT * 128, _CHUNK_IDX), pad_blk], axis=1)
    out1d = _sc_call(lines, idx2d)
    o = out1d.reshape(2, _B, _TPC * _D)
    return jnp.concatenate([o[0], o[1]], axis=1)


# speedup vs baseline: 4.7866x; 1.0712x over previous
import functools

import jax
import jax.numpy as jnp
from jax import lax
from jax.experimental import pallas as pl
from jax.experimental.pallas import tpu as pltpu
from jax.experimental.pallas import tpu_sc as plsc

_NUM_TABLES = 26
_VOCAB = 100000
_EMBED_DIM = 32
_BATCH = 1024
_HIST = 20

_NUM_WORKERS = 32
_BATCH_PER_WORKER = _BATCH // _NUM_WORKERS
_ROWS_PER_CHUNK = _BATCH_PER_WORKER * _HIST
_OUT_COLS = _NUM_TABLES * _EMBED_DIM
_INV_HIST = 1.0 / _HIST


def _sc_body(tab_hbm, idx_hbm, out_hbm, idx_v, rows_v, out_v, sem0, sem1, isem):
    wid = lax.axis_index("s") * 2 + lax.axis_index("c")
    gather_sems = (sem0, sem1)

    def idx_load(t):
        return pltpu.async_copy(
            idx_hbm.at[t * _NUM_WORKERS + wid], idx_v.at[t % 2], isem
        )

    def fire(t):
        buf = t % 2
        return pltpu.async_copy(
            tab_hbm.at[idx_v.at[buf]], rows_v.at[buf], gather_sems[buf]
        )

    def reduce_chunk(t):
        buf = t % 2
        col = t * _EMBED_DIM

        def bag_body(j, carry):
            r0 = j * _HIST
            acc_lo = rows_v[buf, r0, pl.ds(0, 16)]
            acc_hi = rows_v[buf, r0, pl.ds(16, 16)]
            for h in range(1, _HIST):
                acc_lo = acc_lo + rows_v[buf, r0 + h, pl.ds(0, 16)]
                acc_hi = acc_hi + rows_v[buf, r0 + h, pl.ds(16, 16)]
            out_v[j, pl.ds(col, 16)] = acc_lo * _INV_HIST
            out_v[j, pl.ds(col + 16, 16)] = acc_hi * _INV_HIST
            return carry

        lax.fori_loop(0, _BATCH_PER_WORKER, bag_body, 0)

    idx_copies = [None, None]
    gather_copies = [None, None]
    idx_copies[0] = idx_load(0)
    for t in range(_NUM_TABLES + 1):
        if t < _NUM_TABLES:
            idx_copies[t % 2].wait()
            gather_copies[t % 2] = fire(t)
        if t >= 1:
            gather_copies[(t - 1) % 2].wait()
            reduce_chunk(t - 1)
        if t + 1 < _NUM_TABLES:
            idx_copies[(t + 1) % 2] = idx_load(t + 1)

    pltpu.sync_copy(out_v, out_hbm.at[pl.ds(wid * _BATCH_PER_WORKER, _BATCH_PER_WORKER)])


_sc_lookup = functools.partial(
    pl.kernel,
    out_type=jax.ShapeDtypeStruct((_BATCH, _OUT_COLS), jnp.float32),
    mesh=plsc.VectorSubcoreMesh(core_axis_name="c", subcore_axis_name="s"),
    scratch_types=[
        pltpu.VMEM((2, _ROWS_PER_CHUNK), jnp.int32),
        pltpu.VMEM((2, _ROWS_PER_CHUNK, _EMBED_DIM), jnp.float32),
        pltpu.VMEM((_BATCH_PER_WORKER, _OUT_COLS), jnp.float32),
        pltpu.SemaphoreType.DMA,
        pltpu.SemaphoreType.DMA,
        pltpu.SemaphoreType.DMA,
    ],
    compiler_params=pltpu.CompilerParams(use_tc_tiling_on_sc=False),
)(_sc_body)


@jax.jit
def kernel(indices, offsets, tables):
    del offsets
    flat_tables = tables.reshape(_NUM_TABLES * _VOCAB, _EMBED_DIM)
    idx2 = indices.reshape(_NUM_TABLES * _NUM_WORKERS, _ROWS_PER_CHUNK)
    return _sc_lookup(flat_tables, idx2)
